# baseline (speedup 1.0000x reference)
import functools
import numpy as np
import jax
import jax.numpy as jnp
from jax import lax
from jax.experimental import pallas as pl
from jax.experimental.pallas import tpu as pltpu
from jax.experimental.pallas import tpu_sc as plsc
from math import factorial, sqrt

_METADATA_IN = [128, 64, 32]
_LMAX_IN = 2
_LMAX_OUT = 4
_IN_OFF = [0, 128, 320]


def _cg_complex(j1, m1, j2, m2, j3, m3):
    if m1 + m2 != m3:
        return 0.0
    if not (abs(j1 - j2) <= j3 <= j1 + j2):
        return 0.0
    f = factorial
    pre = ((2 * j3 + 1) * f(j1 + j2 - j3) * f(j1 - j2 + j3) * f(-j1 + j2 + j3) / f(j1 + j2 + j3 + 1)) ** 0.5
    pre *= (f(j3 + m3) * f(j3 - m3) * f(j1 + m1) * f(j1 - m1) * f(j2 + m2) * f(j2 - m2)) ** 0.5
    kmin = max(0, j2 - j3 - m1, j1 - j3 + m2)
    kmax = min(j1 + j2 - j3, j1 - m1, j2 + m2)
    s = 0.0
    for k in range(kmin, kmax + 1):
        s += (-1) ** k / (f(k) * f(j1 + j2 - j3 - k) * f(j1 - m1 - k) * f(j2 + m2 - k) * f(j3 - j2 + m1 + k) * f(j3 - j1 - m2 + k))
    return pre * s


def _u_matrix(l):
    U = np.zeros((2 * l + 1, 2 * l + 1), dtype=np.complex128)
    U[l, l] = 1.0
    for m in range(1, l + 1):
        U[l + m, l + m] = (-1) ** m / sqrt(2.0)
        U[l + m, l - m] = 1.0 / sqrt(2.0)
        U[l - m, l - m] = 1j / sqrt(2.0)
        U[l - m, l + m] = -1j * (-1) ** m / sqrt(2.0)
    return U


def _real_cg(l1, l2, l3):
    Cc = np.zeros((2 * l3 + 1, 2 * l1 + 1, 2 * l2 + 1), dtype=np.complex128)
    for m3 in range(-l3, l3 + 1):
        for m1 in range(-l1, l1 + 1):
            m2 = m3 - m1
            if abs(m2) <= l2:
                Cc[m3 + l3, m1 + l1, m2 + l2] = _cg_complex(l1, m1, l2, m2, l3, m3)
    U1, U2, U3 = _u_matrix(l1), _u_matrix(l2), _u_matrix(l3)
    Cr = np.einsum('Km,kij,Ii,Jj->KIJ', U3, Cc, U1.conj(), U2.conj(), optimize=True)
    if np.abs(Cr.imag).max() > np.abs(Cr.real).max():
        return np.ascontiguousarray(Cr.imag)
    return np.ascontiguousarray(Cr.real)


def _build_terms():
    couplings = []
    for lout in range(_LMAX_OUT + 1):
        for l1 in range(_LMAX_IN + 1):
            for l2 in range(_LMAX_IN + 1):
                if abs(l1 - l2) <= lout <= l1 + l2:
                    deg = min(_METADATA_IN[l1], _METADATA_IN[l2])
                    if deg > 0:
                        couplings.append((lout, l1, l2, deg))
    metadata_cg = [0] * (_LMAX_OUT + 1)
    metadata_out = [0] * (_LMAX_OUT + 1)
    for lo, _, _, d in couplings:
        metadata_cg[lo] += d
        metadata_out[lo] = max(metadata_out[lo], d)
    base = np.concatenate([[0], np.cumsum(metadata_cg)[:-1]]).astype(int)
    within = [0] * (_LMAX_OUT + 1)
    terms = []
    for lo, l1, l2, deg in couplings:
        C = _real_cg(l1, l2, lo)
        nz = []
        for k in range(C.shape[0]):
            for i in range(C.shape[1]):
                for j in range(C.shape[2]):
                    v = float(C[k, i, j])
                    if abs(v) > 1e-14:
                        nz.append((k, i, j, v))
        mc_off = int(base[lo]) + within[lo]
        terms.append((lo, l1, l2, deg, mc_off, nz))
        within[lo] += deg
    return terms, metadata_out


_TERMS, _META_OUT = _build_terms()
_DIM_IN = sum((2 * l + 1) * n for l, n in enumerate(_METADATA_IN))
_DIM_OUT = sum((2 * lo + 1) * _META_OUT[lo] for lo in range(_LMAX_OUT + 1))


def _body(x_ref, kc_ref, mc_ref, o_ref):
    xt = x_ref[...].T
    kc = kc_ref[...]
    mc = mc_ref[...]
    nb = xt.shape[1]

    def xseg(l, m, w):
        base = _IN_OFF[l] + m * _METADATA_IN[l]
        return xt[base:base + w, :]

    prods = {}

    def prod(l1, l2, i, j, w):
        key = (l1, l2, i, j) if (l1, l2, i, j) <= (l2, l1, j, i) else (l2, l1, j, i)
        if key not in prods:
            prods[key] = xseg(key[0], key[2], w) * xseg(key[1], key[3], w)
        return prods[key]

    acc = {}

    def add(lo, k, w, arr):
        d = acc.setdefault((lo, k), {})
        d[w] = d[w] + arr if w in d else arr

    ch = 0
    for l, nc in enumerate(_METADATA_IN):
        cp = min(nc, _META_OUT[l])
        kcv = kc[ch:ch + cp, :]
        for m in range(2 * l + 1):
            add(l, m, cp, xseg(l, m, cp) * kcv)
        ch += nc

    for (lo, l1, l2, deg, mc_off, nzs) in _TERMS:
        for (k, i, j, v) in nzs:
            tv = mc[mc_off:mc_off + deg, :] * (0.5 * v)
            add(lo, k, deg, prod(l1, l2, i, j, deg) * tv)

    blocks = []
    for lo in range(_LMAX_OUT + 1):
        W = _META_OUT[lo]
        for k in range(2 * lo + 1):
            d = acc.get((lo, k), {})
            widths = sorted(d, reverse=True)
            if widths and widths[0] == W:
                cur = d[W]
                widths = widths[1:]
            else:
                cur = jnp.zeros((W, nb), xt.dtype)
            for w in widths:
                cur = jnp.concatenate([cur[:w, :] + d[w], cur[w:, :]], axis=0)
            blocks.append(cur)
    o_ref[...] = jnp.concatenate(blocks, axis=0).T


def _tc_call(x, kc, mc, NB=2048):
    n = x.shape[0]
    grid = (n // NB,)
    return pl.pallas_call(
        _body,
        grid=grid,
        compiler_params=pltpu.CompilerParams(
            dimension_semantics=("arbitrary",),
        ),
        in_specs=[
            pl.BlockSpec((NB, _DIM_IN), lambda i: (i, 0)),
            pl.BlockSpec((224, 1), lambda i: (0, 0)),
            pl.BlockSpec((864, 1), lambda i: (0, 0)),
        ],
        out_specs=pl.BlockSpec((NB, _DIM_OUT), lambda i: (i, 0)),
        out_shape=jax.ShapeDtypeStruct((n, _DIM_OUT), x.dtype),
    )(x, kc, mc)


_NW = 32
_L = 16


def _norm_key(l1, l2, i, j):
    return (l1, l2, i, j) if (l1, l2, i, j) <= (l2, l1, j, i) else (l2, l1, j, i)


def _build_sc_schedule():
    uses = {}
    for (lo, l1, l2, deg, mc_off, nzs) in _TERMS:
        for (k, i, j, v) in nzs:
            key = _norm_key(l1, l2, i, j)
            uses[key] = uses.get(key, 0) + 1
    prod_base = {}
    tot = 0
    for (lo, l1, l2, deg, mc_off, nzs) in _TERMS:
        for (k, i, j, v) in nzs:
            key = _norm_key(l1, l2, i, j)
            if key not in prod_base:
                prod_base[key] = tot
                tot += deg
    def xoff(l, m):
        return _IN_OFF[l] + m * _METADATA_IN[l]
    p1 = []
    for (l1, l2, i, j), b in prod_base.items():
        deg = min(_METADATA_IN[l1], _METADATA_IN[l2])
        for cc in range(deg // _L):
            p1.append((b + cc * _L, xoff(l1, i) + cc * _L, xoff(l2, j) + cc * _L))
    out_off = {}
    off = 0
    for lo in range(_LMAX_OUT + 1):
        W = _META_OUT[lo]
        for k in range(2 * lo + 1):
            out_off[(lo, k)] = off
            off += W
    ochunks = {}
    tcoef = []
    for (lo, l1, l2, deg, mc_off, nzs) in _TERMS:
        for (k, i, j, v) in nzs:
            key = _norm_key(l1, l2, i, j)
            for cc in range(deg // _L):
                ci = len(tcoef) * _L
                tcoef.append((mc_off + cc * _L, 0.5 * v))
                if key in prod_base:
                    op = ('p', prod_base[key] + cc * _L, ci)
                else:
                    op = ('x', xoff(l1, i) + cc * _L, xoff(l2, j) + cc * _L, ci)
                ochunks.setdefault((lo, k, cc), []).append(op)
    ch = 0
    for l, nc in enumerate(_METADATA_IN):
        cp = min(nc, _META_OUT[l])
        for m in range(2 * l + 1):
            for cc in range(cp // _L):
                ochunks.setdefault((l, m, cc), []).append(
                    ('kp', xoff(l, m) + cc * _L, ch + cc * _L))
        ch += nc
    sched = []
    for lo in range(_LMAX_OUT + 1):
        W = _META_OUT[lo]
        for k in range(2 * lo + 1):
            for cc in range(W // _L):
                sched.append((out_off[(lo, k)] + cc * _L,
                              ochunks.get((lo, k, cc), [])))
    return p1, sched, tot, tcoef


_P1, _SCHED, _PROD_TOT, _TCOEF = _build_sc_schedule()


def _sc_call(x, kc, mc, CN=4):
    n = x.shape[0]
    per_w = n // _NW
    nsteps = per_w // CN
    assert per_w % CN == 0
    ntc = len(_TCOEF) * _L
    mesh = plsc.VectorSubcoreMesh(core_axis_name="c", subcore_axis_name="s")

    @functools.partial(
        pl.kernel,
        out_type=jax.ShapeDtypeStruct((n, _DIM_OUT), jnp.float32),
        mesh=mesh,
        scratch_types=[
            pltpu.VMEM((CN, _DIM_IN), jnp.float32),
            pltpu.VMEM((CN, _DIM_OUT), jnp.float32),
            pltpu.VMEM((224,), jnp.float32),
            pltpu.VMEM((864,), jnp.float32),
            pltpu.VMEM((_PROD_TOT,), jnp.float32),
            pltpu.VMEM((ntc,), jnp.float32),
        ],
    )
    def sc_kernel(x_hbm, kc_hbm, mc_hbm, out_hbm,
                  xbuf0, obuf0, kcbuf, mcbuf, pbuf, tcoef):
        wid = lax.axis_index("s") * 2 + lax.axis_index("c")
        base = wid * per_w
        pltpu.sync_copy(kc_hbm, kcbuf)
        pltpu.sync_copy(mc_hbm, mcbuf)
        for idx, (mo, s) in enumerate(_TCOEF):
            tcoef[pl.ds(idx * _L, _L)] = mcbuf[pl.ds(mo, _L)] * s

        def compute(xbuf, obuf):
            for nl in range(CN):
                for (pb, o1, o2) in _P1:
                    pbuf[pl.ds(pb, _L)] = (xbuf[nl, pl.ds(o1, _L)]
                                           * xbuf[nl, pl.ds(o2, _L)])
                for (oo, ops) in _SCHED:
                    acc = None
                    for op in ops:
                        if op[0] == 'p':
                            _, pb, ci = op
                            c = pbuf[pl.ds(pb, _L)] * tcoef[pl.ds(ci, _L)]
                        elif op[0] == 'x':
                            _, o1, o2, ci = op
                            c = (xbuf[nl, pl.ds(o1, _L)]
                                 * xbuf[nl, pl.ds(o2, _L)]
                                 * tcoef[pl.ds(ci, _L)])
                        else:
                            _, xo, ko = op
                            c = xbuf[nl, pl.ds(xo, _L)] * kcbuf[pl.ds(ko, _L)]
                        acc = c if acc is None else acc + c
                    if acc is None:
                        acc = jnp.zeros((_L,), jnp.float32)
                    obuf[nl, pl.ds(oo, _L)] = acc

        def step(it, carry):
            nb = base + it * CN
            pltpu.sync_copy(x_hbm.at[pl.ds(nb, CN)], xbuf0)
            compute(xbuf0, obuf0)
            pltpu.sync_copy(obuf0, out_hbm.at[pl.ds(nb, CN)])
            return carry

        lax.fori_loop(0, nsteps, step, 0)

    return sc_kernel(x, kc, mc)


def kernel(x, keep_coeff, mix_coeff):
    kc = keep_coeff.reshape(-1, 1)
    mc = mix_coeff.reshape(-1, 1)
    return _sc_call(x, keep_coeff, mix_coeff)


# speedup vs baseline: 1.3081x; 1.3081x over previous
import functools
import numpy as np
import jax
import jax.numpy as jnp
from jax import lax
from jax.experimental import pallas as pl
from jax.experimental.pallas import tpu as pltpu
from jax.experimental.pallas import tpu_sc as plsc
from math import factorial, sqrt

_METADATA_IN = [128, 64, 32]
_LMAX_IN = 2
_LMAX_OUT = 4
_IN_OFF = [0, 128, 320]


def _cg_complex(j1, m1, j2, m2, j3, m3):
    if m1 + m2 != m3:
        return 0.0
    if not (abs(j1 - j2) <= j3 <= j1 + j2):
        return 0.0
    f = factorial
    pre = ((2 * j3 + 1) * f(j1 + j2 - j3) * f(j1 - j2 + j3) * f(-j1 + j2 + j3) / f(j1 + j2 + j3 + 1)) ** 0.5
    pre *= (f(j3 + m3) * f(j3 - m3) * f(j1 + m1) * f(j1 - m1) * f(j2 + m2) * f(j2 - m2)) ** 0.5
    kmin = max(0, j2 - j3 - m1, j1 - j3 + m2)
    kmax = min(j1 + j2 - j3, j1 - m1, j2 + m2)
    s = 0.0
    for k in range(kmin, kmax + 1):
        s += (-1) ** k / (f(k) * f(j1 + j2 - j3 - k) * f(j1 - m1 - k) * f(j2 + m2 - k) * f(j3 - j2 + m1 + k) * f(j3 - j1 - m2 + k))
    return pre * s


def _u_matrix(l):
    U = np.zeros((2 * l + 1, 2 * l + 1), dtype=np.complex128)
    U[l, l] = 1.0
    for m in range(1, l + 1):
        U[l + m, l + m] = (-1) ** m / sqrt(2.0)
        U[l + m, l - m] = 1.0 / sqrt(2.0)
        U[l - m, l - m] = 1j / sqrt(2.0)
        U[l - m, l + m] = -1j * (-1) ** m / sqrt(2.0)
    return U


def _real_cg(l1, l2, l3):
    Cc = np.zeros((2 * l3 + 1, 2 * l1 + 1, 2 * l2 + 1), dtype=np.complex128)
    for m3 in range(-l3, l3 + 1):
        for m1 in range(-l1, l1 + 1):
            m2 = m3 - m1
            if abs(m2) <= l2:
                Cc[m3 + l3, m1 + l1, m2 + l2] = _cg_complex(l1, m1, l2, m2, l3, m3)
    U1, U2, U3 = _u_matrix(l1), _u_matrix(l2), _u_matrix(l3)
    Cr = np.einsum('Km,kij,Ii,Jj->KIJ', U3, Cc, U1.conj(), U2.conj(), optimize=True)
    if np.abs(Cr.imag).max() > np.abs(Cr.real).max():
        return np.ascontiguousarray(Cr.imag)
    return np.ascontiguousarray(Cr.real)


def _build_terms():
    couplings = []
    for lout in range(_LMAX_OUT + 1):
        for l1 in range(_LMAX_IN + 1):
            for l2 in range(_LMAX_IN + 1):
                if abs(l1 - l2) <= lout <= l1 + l2:
                    deg = min(_METADATA_IN[l1], _METADATA_IN[l2])
                    if deg > 0:
                        couplings.append((lout, l1, l2, deg))
    metadata_cg = [0] * (_LMAX_OUT + 1)
    metadata_out = [0] * (_LMAX_OUT + 1)
    for lo, _, _, d in couplings:
        metadata_cg[lo] += d
        metadata_out[lo] = max(metadata_out[lo], d)
    base = np.concatenate([[0], np.cumsum(metadata_cg)[:-1]]).astype(int)
    within = [0] * (_LMAX_OUT + 1)
    terms = []
    for lo, l1, l2, deg in couplings:
        C = _real_cg(l1, l2, lo)
        nz = []
        for k in range(C.shape[0]):
            for i in range(C.shape[1]):
                for j in range(C.shape[2]):
                    v = float(C[k, i, j])
                    if abs(v) > 1e-14:
                        nz.append((k, i, j, v))
        mc_off = int(base[lo]) + within[lo]
        terms.append((lo, l1, l2, deg, mc_off, nz))
        within[lo] += deg
    return terms, metadata_out


_TERMS, _META_OUT = _build_terms()
_DIM_IN = sum((2 * l + 1) * n for l, n in enumerate(_METADATA_IN))
_DIM_OUT = sum((2 * lo + 1) * _META_OUT[lo] for lo in range(_LMAX_OUT + 1))


def _body(x_ref, kc_ref, mc_ref, o_ref):
    xt = x_ref[...].T
    kc = kc_ref[...]
    mc = mc_ref[...]
    nb = xt.shape[1]

    def xseg(l, m, w):
        base = _IN_OFF[l] + m * _METADATA_IN[l]
        return xt[base:base + w, :]

    prods = {}

    def prod(l1, l2, i, j, w):
        key = (l1, l2, i, j) if (l1, l2, i, j) <= (l2, l1, j, i) else (l2, l1, j, i)
        if key not in prods:
            prods[key] = xseg(key[0], key[2], w) * xseg(key[1], key[3], w)
        return prods[key]

    acc = {}

    def add(lo, k, w, arr):
        d = acc.setdefault((lo, k), {})
        d[w] = d[w] + arr if w in d else arr

    ch = 0
    for l, nc in enumerate(_METADATA_IN):
        cp = min(nc, _META_OUT[l])
        kcv = kc[ch:ch + cp, :]
        for m in range(2 * l + 1):
            add(l, m, cp, xseg(l, m, cp) * kcv)
        ch += nc

    for (lo, l1, l2, deg, mc_off, nzs) in _TERMS:
        for (k, i, j, v) in nzs:
            tv = mc[mc_off:mc_off + deg, :] * (0.5 * v)
            add(lo, k, deg, prod(l1, l2, i, j, deg) * tv)

    blocks = []
    for lo in range(_LMAX_OUT + 1):
        W = _META_OUT[lo]
        for k in range(2 * lo + 1):
            d = acc.get((lo, k), {})
            widths = sorted(d, reverse=True)
            if widths and widths[0] == W:
                cur = d[W]
                widths = widths[1:]
            else:
                cur = jnp.zeros((W, nb), xt.dtype)
            for w in widths:
                cur = jnp.concatenate([cur[:w, :] + d[w], cur[w:, :]], axis=0)
            blocks.append(cur)
    o_ref[...] = jnp.concatenate(blocks, axis=0).T


def _tc_call(x, kc, mc, NB=2048):
    n = x.shape[0]
    grid = (n // NB,)
    return pl.pallas_call(
        _body,
        grid=grid,
        compiler_params=pltpu.CompilerParams(
            dimension_semantics=("arbitrary",),
        ),
        in_specs=[
            pl.BlockSpec((NB, _DIM_IN), lambda i: (i, 0)),
            pl.BlockSpec((224, 1), lambda i: (0, 0)),
            pl.BlockSpec((864, 1), lambda i: (0, 0)),
        ],
        out_specs=pl.BlockSpec((NB, _DIM_OUT), lambda i: (i, 0)),
        out_shape=jax.ShapeDtypeStruct((n, _DIM_OUT), x.dtype),
    )(x, kc, mc)


_NW = 32
_L = 16


def _norm_key(l1, l2, i, j):
    return (l1, l2, i, j) if (l1, l2, i, j) <= (l2, l1, j, i) else (l2, l1, j, i)


def _build_sc_schedule():
    uses = {}
    for (lo, l1, l2, deg, mc_off, nzs) in _TERMS:
        for (k, i, j, v) in nzs:
            key = _norm_key(l1, l2, i, j)
            uses[key] = uses.get(key, 0) + 1
    prod_base = {}
    tot = 0
    for (lo, l1, l2, deg, mc_off, nzs) in _TERMS:
        for (k, i, j, v) in nzs:
            key = _norm_key(l1, l2, i, j)
            if key not in prod_base:
                prod_base[key] = tot
                tot += deg
    def xoff(l, m):
        return _IN_OFF[l] + m * _METADATA_IN[l]
    p1 = []
    for (l1, l2, i, j), b in prod_base.items():
        deg = min(_METADATA_IN[l1], _METADATA_IN[l2])
        for cc in range(deg // _L):
            p1.append((b + cc * _L, xoff(l1, i) + cc * _L, xoff(l2, j) + cc * _L))
    out_off = {}
    off = 0
    for lo in range(_LMAX_OUT + 1):
        W = _META_OUT[lo]
        for k in range(2 * lo + 1):
            out_off[(lo, k)] = off
            off += W
    ochunks = {}
    tcoef = []
    for (lo, l1, l2, deg, mc_off, nzs) in _TERMS:
        for (k, i, j, v) in nzs:
            key = _norm_key(l1, l2, i, j)
            for cc in range(deg // _L):
                ci = len(tcoef) * _L
                tcoef.append((mc_off + cc * _L, 0.5 * v))
                if key in prod_base:
                    op = ('p', prod_base[key] + cc * _L, ci)
                else:
                    op = ('x', xoff(l1, i) + cc * _L, xoff(l2, j) + cc * _L, ci)
                ochunks.setdefault((lo, k, cc), []).append(op)
    ch = 0
    for l, nc in enumerate(_METADATA_IN):
        cp = min(nc, _META_OUT[l])
        for m in range(2 * l + 1):
            for cc in range(cp // _L):
                ochunks.setdefault((l, m, cc), []).append(
                    ('kp', xoff(l, m) + cc * _L, ch + cc * _L))
        ch += nc
    sched = []
    for lo in range(_LMAX_OUT + 1):
        W = _META_OUT[lo]
        for k in range(2 * lo + 1):
            for cc in range(W // _L):
                sched.append((out_off[(lo, k)] + cc * _L,
                              ochunks.get((lo, k, cc), [])))
    return p1, sched, tot, tcoef


_P1, _SCHED, _PROD_TOT, _TCOEF = _build_sc_schedule()


def _sc_call(x, kc, mc):
    n = x.shape[0]
    per_w = n // _NW
    nsteps = per_w
    assert nsteps % 2 == 0
    ntc = len(_TCOEF) * _L
    mesh = plsc.VectorSubcoreMesh(core_axis_name="c", subcore_axis_name="s")

    @functools.partial(
        pl.kernel,
        out_type=jax.ShapeDtypeStruct((n, _DIM_OUT), jnp.float32),
        mesh=mesh,
        scratch_types=[
            pltpu.VMEM((1, _DIM_IN), jnp.float32),
            pltpu.VMEM((1, _DIM_IN), jnp.float32),
            pltpu.VMEM((1, _DIM_OUT), jnp.float32),
            pltpu.VMEM((1, _DIM_OUT), jnp.float32),
            pltpu.VMEM((224,), jnp.float32),
            pltpu.VMEM((864,), jnp.float32),
            pltpu.VMEM((_PROD_TOT,), jnp.float32),
            pltpu.VMEM((ntc,), jnp.float32),
            pltpu.SemaphoreType.DMA,
            pltpu.SemaphoreType.DMA,
            pltpu.SemaphoreType.DMA,
            pltpu.SemaphoreType.DMA,
        ],
    )
    def sc_kernel(x_hbm, kc_hbm, mc_hbm, out_hbm,
                  xbuf0, xbuf1, obuf0, obuf1, kcbuf, mcbuf, pbuf, tcoef,
                  semx0, semx1, semo0, semo1):
        wid = lax.axis_index("s") * 2 + lax.axis_index("c")
        base = wid * per_w
        pltpu.sync_copy(kc_hbm, kcbuf)
        pltpu.sync_copy(mc_hbm, mcbuf)
        for idx, (mo, s) in enumerate(_TCOEF):
            tcoef[pl.ds(idx * _L, _L)] = mcbuf[pl.ds(mo, _L)] * s

        xbufs = (xbuf0, xbuf1)
        obufs = (obuf0, obuf1)
        semxs = (semx0, semx1)
        semos = (semo0, semo1)

        def compute(xbuf, obuf):
            for (pb, o1, o2) in _P1:
                pbuf[pl.ds(pb, _L)] = (xbuf[0, pl.ds(o1, _L)]
                                       * xbuf[0, pl.ds(o2, _L)])
            for (oo, ops) in _SCHED:
                acc = None
                for op in ops:
                    if op[0] == 'p':
                        _, pb, ci = op
                        c = pbuf[pl.ds(pb, _L)] * tcoef[pl.ds(ci, _L)]
                    elif op[0] == 'x':
                        _, o1, o2, ci = op
                        c = (xbuf[0, pl.ds(o1, _L)]
                             * xbuf[0, pl.ds(o2, _L)]
                             * tcoef[pl.ds(ci, _L)])
                    else:
                        _, xo, ko = op
                        c = xbuf[0, pl.ds(xo, _L)] * kcbuf[pl.ds(ko, _L)]
                    acc = c if acc is None else acc + c
                if acc is None:
                    acc = jnp.zeros((_L,), jnp.float32)
                obuf[0, pl.ds(oo, _L)] = acc

        pltpu.async_copy(x_hbm.at[pl.ds(base, 1)], xbuf0, semx0)
        pltpu.async_copy(x_hbm.at[pl.ds(base + 1, 1)], xbuf1, semx1)

        def step(g, carry):
            for par in (0, 1):
                it = g * 2 + par
                nb = base + it
                pltpu.make_async_copy(x_hbm.at[pl.ds(nb, 1)],
                                      xbufs[par], semxs[par]).wait()

                @pl.when(g > 0)
                def _wait_out():
                    pltpu.make_async_copy(obufs[par],
                                          out_hbm.at[pl.ds(nb, 1)],
                                          semos[par]).wait()

                compute(xbufs[par], obufs[par])
                pltpu.async_copy(obufs[par], out_hbm.at[pl.ds(nb, 1)],
                                 semos[par])
                nb2 = base + jnp.minimum(it + 2, nsteps - 1)
                pltpu.async_copy(x_hbm.at[pl.ds(nb2, 1)],
                                 xbufs[par], semxs[par])
            return carry

        lax.fori_loop(0, nsteps // 2, step, 0)
        pltpu.make_async_copy(x_hbm.at[pl.ds(base, 1)], xbuf0, semx0).wait()
        pltpu.make_async_copy(x_hbm.at[pl.ds(base, 1)], xbuf1, semx1).wait()
        pltpu.make_async_copy(obuf0, out_hbm.at[pl.ds(base, 1)], semo0).wait()
        pltpu.make_async_copy(obuf1, out_hbm.at[pl.ds(base, 1)], semo1).wait()

    return sc_kernel(x, kc, mc)


def kernel(x, keep_coeff, mix_coeff):
    kc = keep_coeff.reshape(-1, 1)
    mc = mix_coeff.reshape(-1, 1)
    return _sc_call(x, keep_coeff, mix_coeff)


# speedup vs baseline: 9.3928x; 7.1806x over previous
import functools
import numpy as np
import jax
import jax.numpy as jnp
from jax import lax
from jax.experimental import pallas as pl
from jax.experimental.pallas import tpu as pltpu
from jax.experimental.pallas import tpu_sc as plsc
from math import factorial, sqrt

_METADATA_IN = [128, 64, 32]
_LMAX_IN = 2
_LMAX_OUT = 4
_IN_OFF = [0, 128, 320]


def _cg_complex(j1, m1, j2, m2, j3, m3):
    if m1 + m2 != m3:
        return 0.0
    if not (abs(j1 - j2) <= j3 <= j1 + j2):
        return 0.0
    f = factorial
    pre = ((2 * j3 + 1) * f(j1 + j2 - j3) * f(j1 - j2 + j3) * f(-j1 + j2 + j3) / f(j1 + j2 + j3 + 1)) ** 0.5
    pre *= (f(j3 + m3) * f(j3 - m3) * f(j1 + m1) * f(j1 - m1) * f(j2 + m2) * f(j2 - m2)) ** 0.5
    kmin = max(0, j2 - j3 - m1, j1 - j3 + m2)
    kmax = min(j1 + j2 - j3, j1 - m1, j2 + m2)
    s = 0.0
    for k in range(kmin, kmax + 1):
        s += (-1) ** k / (f(k) * f(j1 + j2 - j3 - k) * f(j1 - m1 - k) * f(j2 + m2 - k) * f(j3 - j2 + m1 + k) * f(j3 - j1 - m2 + k))
    return pre * s


def _u_matrix(l):
    U = np.zeros((2 * l + 1, 2 * l + 1), dtype=np.complex128)
    U[l, l] = 1.0
    for m in range(1, l + 1):
        U[l + m, l + m] = (-1) ** m / sqrt(2.0)
        U[l + m, l - m] = 1.0 / sqrt(2.0)
        U[l - m, l - m] = 1j / sqrt(2.0)
        U[l - m, l + m] = -1j * (-1) ** m / sqrt(2.0)
    return U


def _real_cg(l1, l2, l3):
    Cc = np.zeros((2 * l3 + 1, 2 * l1 + 1, 2 * l2 + 1), dtype=np.complex128)
    for m3 in range(-l3, l3 + 1):
        for m1 in range(-l1, l1 + 1):
            m2 = m3 - m1
            if abs(m2) <= l2:
                Cc[m3 + l3, m1 + l1, m2 + l2] = _cg_complex(l1, m1, l2, m2, l3, m3)
    U1, U2, U3 = _u_matrix(l1), _u_matrix(l2), _u_matrix(l3)
    Cr = np.einsum('Km,kij,Ii,Jj->KIJ', U3, Cc, U1.conj(), U2.conj(), optimize=True)
    if np.abs(Cr.imag).max() > np.abs(Cr.real).max():
        return np.ascontiguousarray(Cr.imag)
    return np.ascontiguousarray(Cr.real)


def _build_terms():
    couplings = []
    for lout in range(_LMAX_OUT + 1):
        for l1 in range(_LMAX_IN + 1):
            for l2 in range(_LMAX_IN + 1):
                if abs(l1 - l2) <= lout <= l1 + l2:
                    deg = min(_METADATA_IN[l1], _METADATA_IN[l2])
                    if deg > 0:
                        couplings.append((lout, l1, l2, deg))
    metadata_cg = [0] * (_LMAX_OUT + 1)
    metadata_out = [0] * (_LMAX_OUT + 1)
    for lo, _, _, d in couplings:
        metadata_cg[lo] += d
        metadata_out[lo] = max(metadata_out[lo], d)
    base = np.concatenate([[0], np.cumsum(metadata_cg)[:-1]]).astype(int)
    within = [0] * (_LMAX_OUT + 1)
    terms = []
    for lo, l1, l2, deg in couplings:
        C = _real_cg(l1, l2, lo)
        nz = []
        for k in range(C.shape[0]):
            for i in range(C.shape[1]):
                for j in range(C.shape[2]):
                    v = float(C[k, i, j])
                    if abs(v) > 1e-14:
                        nz.append((k, i, j, v))
        mc_off = int(base[lo]) + within[lo]
        terms.append((lo, l1, l2, deg, mc_off, nz))
        within[lo] += deg
    return terms, metadata_out


_TERMS, _META_OUT = _build_terms()
_DIM_IN = sum((2 * l + 1) * n for l, n in enumerate(_METADATA_IN))
_DIM_OUT = sum((2 * lo + 1) * _META_OUT[lo] for lo in range(_LMAX_OUT + 1))


def _body(x_ref, kc_ref, mc_ref, o_ref):
    xt = x_ref[...].T
    kc = kc_ref[...]
    mc = mc_ref[...]
    nb = xt.shape[1]

    def xseg(l, m, w):
        base = _IN_OFF[l] + m * _METADATA_IN[l]
        return xt[base:base + w, :]

    prods = {}

    def prod(l1, l2, i, j, w):
        key = (l1, l2, i, j) if (l1, l2, i, j) <= (l2, l1, j, i) else (l2, l1, j, i)
        if key not in prods:
            prods[key] = xseg(key[0], key[2], w) * xseg(key[1], key[3], w)
        return prods[key]

    acc = {}

    def add(lo, k, w, arr):
        d = acc.setdefault((lo, k), {})
        d[w] = d[w] + arr if w in d else arr

    ch = 0
    for l, nc in enumerate(_METADATA_IN):
        cp = min(nc, _META_OUT[l])
        kcv = kc[ch:ch + cp, :]
        for m in range(2 * l + 1):
            add(l, m, cp, xseg(l, m, cp) * kcv)
        ch += nc

    for (lo, l1, l2, deg, mc_off, nzs) in _TERMS:
        for (k, i, j, v) in nzs:
            tv = mc[mc_off:mc_off + deg, :] * (0.5 * v)
            add(lo, k, deg, prod(l1, l2, i, j, deg) * tv)

    blocks = []
    for lo in range(_LMAX_OUT + 1):
        W = _META_OUT[lo]
        for k in range(2 * lo + 1):
            d = acc.get((lo, k), {})
            widths = sorted(d, reverse=True)
            if widths and widths[0] == W:
                cur = d[W]
                widths = widths[1:]
            else:
                cur = jnp.zeros((W, nb), xt.dtype)
            for w in widths:
                cur = jnp.concatenate([cur[:w, :] + d[w], cur[w:, :]], axis=0)
            blocks.append(cur)
    o_ref[...] = jnp.concatenate(blocks, axis=0).T


def _tc_call(x, kc, mc, NB=2048):
    n = x.shape[0]
    grid = (n // NB,)
    return pl.pallas_call(
        _body,
        grid=grid,
        compiler_params=pltpu.CompilerParams(
            dimension_semantics=("arbitrary",),
        ),
        in_specs=[
            pl.BlockSpec((NB, _DIM_IN), lambda i: (i, 0)),
            pl.BlockSpec((224, 1), lambda i: (0, 0)),
            pl.BlockSpec((864, 1), lambda i: (0, 0)),
        ],
        out_specs=pl.BlockSpec((NB, _DIM_OUT), lambda i: (i, 0)),
        out_shape=jax.ShapeDtypeStruct((n, _DIM_OUT), x.dtype),
    )(x, kc, mc)


_NW = 32
_L = 16


def _norm_key(l1, l2, i, j):
    return (l1, l2, i, j) if (l1, l2, i, j) <= (l2, l1, j, i) else (l2, l1, j, i)


def _build_sc_schedule():
    uses = {}
    for (lo, l1, l2, deg, mc_off, nzs) in _TERMS:
        for (k, i, j, v) in nzs:
            key = _norm_key(l1, l2, i, j)
            uses[key] = uses.get(key, 0) + 1
    prod_base = {}
    tot = 0
    for (lo, l1, l2, deg, mc_off, nzs) in _TERMS:
        for (k, i, j, v) in nzs:
            key = _norm_key(l1, l2, i, j)
            if key not in prod_base:
                prod_base[key] = tot
                tot += deg
    def xoff(l, m):
        return _IN_OFF[l] + m * _METADATA_IN[l]
    p1 = []
    for (l1, l2, i, j), b in prod_base.items():
        deg = min(_METADATA_IN[l1], _METADATA_IN[l2])
        for cc in range(deg // _L):
            p1.append((b + cc * _L, xoff(l1, i) + cc * _L, xoff(l2, j) + cc * _L))
    out_off = {}
    off = 0
    for lo in range(_LMAX_OUT + 1):
        W = _META_OUT[lo]
        for k in range(2 * lo + 1):
            out_off[(lo, k)] = off
            off += W
    ochunks = {}
    tcoef = []
    for (lo, l1, l2, deg, mc_off, nzs) in _TERMS:
        for (k, i, j, v) in nzs:
            key = _norm_key(l1, l2, i, j)
            for cc in range(deg // _L):
                ci = len(tcoef) * _L
                tcoef.append((mc_off + cc * _L, 0.5 * v))
                if key in prod_base:
                    op = ('p', prod_base[key] + cc * _L, ci)
                else:
                    op = ('x', xoff(l1, i) + cc * _L, xoff(l2, j) + cc * _L, ci)
                ochunks.setdefault((lo, k, cc), []).append(op)
    ch = 0
    for l, nc in enumerate(_METADATA_IN):
        cp = min(nc, _META_OUT[l])
        for m in range(2 * l + 1):
            for cc in range(cp // _L):
                ochunks.setdefault((l, m, cc), []).append(
                    ('kp', xoff(l, m) + cc * _L, ch + cc * _L))
        ch += nc
    sched = []
    for lo in range(_LMAX_OUT + 1):
        W = _META_OUT[lo]
        for k in range(2 * lo + 1):
            for cc in range(W // _L):
                sched.append((out_off[(lo, k)] + cc * _L,
                              ochunks.get((lo, k, cc), [])))
    return p1, sched, tot, tcoef


_P1, _SCHED, _PROD_TOT, _TCOEF = _build_sc_schedule()


def _sc_call(x, kc, mc):
    n = x.shape[0]
    per_w = n // _NW
    mesh = plsc.VectorSubcoreMesh(core_axis_name="c", subcore_axis_name="s")

    @functools.partial(
        pl.kernel,
        out_type=jax.ShapeDtypeStruct((n, _DIM_OUT), jnp.float32),
        mesh=mesh,
        scratch_types=[
            pltpu.VMEM((1, _DIM_IN), jnp.float32),
            pltpu.VMEM((1, _DIM_OUT), jnp.float32),
            pltpu.VMEM((224,), jnp.float32),
            pltpu.VMEM((864,), jnp.float32),
            pltpu.VMEM((_PROD_TOT,), jnp.float32),
        ],
    )
    def sc_kernel(x_hbm, kc_hbm, mc_hbm, out_hbm, xbuf, obuf, kcbuf, mcbuf, pbuf):
        wid = lax.axis_index("s") * 2 + lax.axis_index("c")
        base = wid * per_w
        pltpu.sync_copy(kc_hbm, kcbuf)
        pltpu.sync_copy(mc_hbm, mcbuf)

        def step(it, carry):
            nb = base + it
            pltpu.sync_copy(x_hbm.at[pl.ds(nb, 1)], xbuf)
            for (pb, o1, o2) in _P1:
                pbuf[pl.ds(pb, _L)] = (xbuf[0, pl.ds(o1, _L)]
                                       * xbuf[0, pl.ds(o2, _L)])
            for (oo, ops) in _SCHED:
                acc = None
                for op in ops:
                    if op[0] == 'p':
                        _, pb, ci = op
                        mo, s = _TCOEF[ci // _L]
                        c = pbuf[pl.ds(pb, _L)] * (mcbuf[pl.ds(mo, _L)] * s)
                    else:
                        _, xo, ko = op
                        c = xbuf[0, pl.ds(xo, _L)] * kcbuf[pl.ds(ko, _L)]
                    acc = c if acc is None else acc + c
                if acc is None:
                    acc = jnp.zeros((_L,), jnp.float32)
                obuf[0, pl.ds(oo, _L)] = acc
            pltpu.sync_copy(obuf, out_hbm.at[pl.ds(nb, 1)])
            return carry

        lax.fori_loop(0, per_w, step, 0)

    return sc_kernel(x, kc, mc)


_N_SC = 2048


def kernel(x, keep_coeff, mix_coeff):
    n = x.shape[0]
    kc2 = keep_coeff.reshape(-1, 1)
    mc2 = mix_coeff.reshape(-1, 1)
    n_tc = n - _N_SC
    out_tc = _tc_call(x[:n_tc], kc2, mc2, NB=1024)
    out_sc = _sc_call(x[n_tc:], keep_coeff, mix_coeff)
    return jnp.concatenate([out_tc, out_sc], axis=0)


# speedup vs baseline: 18.5743x; 1.9775x over previous
import functools
import numpy as np
import jax
import jax.numpy as jnp
from jax import lax
from jax.experimental import pallas as pl
from jax.experimental.pallas import tpu as pltpu
from jax.experimental.pallas import tpu_sc as plsc
from math import factorial, sqrt

_METADATA_IN = [128, 64, 32]
_LMAX_IN = 2
_LMAX_OUT = 4
_IN_OFF = [0, 128, 320]


def _cg_complex(j1, m1, j2, m2, j3, m3):
    if m1 + m2 != m3:
        return 0.0
    if not (abs(j1 - j2) <= j3 <= j1 + j2):
        return 0.0
    f = factorial
    pre = ((2 * j3 + 1) * f(j1 + j2 - j3) * f(j1 - j2 + j3) * f(-j1 + j2 + j3) / f(j1 + j2 + j3 + 1)) ** 0.5
    pre *= (f(j3 + m3) * f(j3 - m3) * f(j1 + m1) * f(j1 - m1) * f(j2 + m2) * f(j2 - m2)) ** 0.5
    kmin = max(0, j2 - j3 - m1, j1 - j3 + m2)
    kmax = min(j1 + j2 - j3, j1 - m1, j2 + m2)
    s = 0.0
    for k in range(kmin, kmax + 1):
        s += (-1) ** k / (f(k) * f(j1 + j2 - j3 - k) * f(j1 - m1 - k) * f(j2 + m2 - k) * f(j3 - j2 + m1 + k) * f(j3 - j1 - m2 + k))
    return pre * s


def _u_matrix(l):
    U = np.zeros((2 * l + 1, 2 * l + 1), dtype=np.complex128)
    U[l, l] = 1.0
    for m in range(1, l + 1):
        U[l + m, l + m] = (-1) ** m / sqrt(2.0)
        U[l + m, l - m] = 1.0 / sqrt(2.0)
        U[l - m, l - m] = 1j / sqrt(2.0)
        U[l - m, l + m] = -1j * (-1) ** m / sqrt(2.0)
    return U


def _real_cg(l1, l2, l3):
    Cc = np.zeros((2 * l3 + 1, 2 * l1 + 1, 2 * l2 + 1), dtype=np.complex128)
    for m3 in range(-l3, l3 + 1):
        for m1 in range(-l1, l1 + 1):
            m2 = m3 - m1
            if abs(m2) <= l2:
                Cc[m3 + l3, m1 + l1, m2 + l2] = _cg_complex(l1, m1, l2, m2, l3, m3)
    U1, U2, U3 = _u_matrix(l1), _u_matrix(l2), _u_matrix(l3)
    Cr = np.einsum('Km,kij,Ii,Jj->KIJ', U3, Cc, U1.conj(), U2.conj(), optimize=True)
    if np.abs(Cr.imag).max() > np.abs(Cr.real).max():
        return np.ascontiguousarray(Cr.imag)
    return np.ascontiguousarray(Cr.real)


def _build_terms():
    couplings = []
    for lout in range(_LMAX_OUT + 1):
        for l1 in range(_LMAX_IN + 1):
            for l2 in range(_LMAX_IN + 1):
                if abs(l1 - l2) <= lout <= l1 + l2:
                    deg = min(_METADATA_IN[l1], _METADATA_IN[l2])
                    if deg > 0:
                        couplings.append((lout, l1, l2, deg))
    metadata_cg = [0] * (_LMAX_OUT + 1)
    metadata_out = [0] * (_LMAX_OUT + 1)
    for lo, _, _, d in couplings:
        metadata_cg[lo] += d
        metadata_out[lo] = max(metadata_out[lo], d)
    base = np.concatenate([[0], np.cumsum(metadata_cg)[:-1]]).astype(int)
    within = [0] * (_LMAX_OUT + 1)
    terms = []
    for lo, l1, l2, deg in couplings:
        C = _real_cg(l1, l2, lo)
        nz = []
        for k in range(C.shape[0]):
            for i in range(C.shape[1]):
                for j in range(C.shape[2]):
                    v = float(C[k, i, j])
                    if abs(v) > 1e-14:
                        nz.append((k, i, j, v))
        mc_off = int(base[lo]) + within[lo]
        terms.append((lo, l1, l2, deg, mc_off, nz))
        within[lo] += deg
    return terms, metadata_out


_TERMS, _META_OUT = _build_terms()
_DIM_IN = sum((2 * l + 1) * n for l, n in enumerate(_METADATA_IN))
_DIM_OUT = sum((2 * lo + 1) * _META_OUT[lo] for lo in range(_LMAX_OUT + 1))


def _body(x_ref, kc_ref, mc_ref, o_ref):
    xt = x_ref[...].T
    kc = kc_ref[...]
    mc = mc_ref[...]
    nb = xt.shape[1]

    def xseg(l, m, w):
        base = _IN_OFF[l] + m * _METADATA_IN[l]
        return xt[base:base + w, :]

    prods = {}

    def prod(l1, l2, i, j, w):
        key = (l1, l2, i, j) if (l1, l2, i, j) <= (l2, l1, j, i) else (l2, l1, j, i)
        if key not in prods:
            prods[key] = xseg(key[0], key[2], w) * xseg(key[1], key[3], w)
        return prods[key]

    acc = {}

    def add(lo, k, w, arr):
        d = acc.setdefault((lo, k), {})
        d[w] = d[w] + arr if w in d else arr

    ch = 0
    for l, nc in enumerate(_METADATA_IN):
        cp = min(nc, _META_OUT[l])
        kcv = kc[ch:ch + cp, :]
        for m in range(2 * l + 1):
            add(l, m, cp, xseg(l, m, cp) * kcv)
        ch += nc

    for (lo, l1, l2, deg, mc_off, nzs) in _TERMS:
        for (k, i, j, v) in nzs:
            tv = mc[mc_off:mc_off + deg, :] * (0.5 * v)
            add(lo, k, deg, prod(l1, l2, i, j, deg) * tv)

    blocks = []
    for lo in range(_LMAX_OUT + 1):
        W = _META_OUT[lo]
        for k in range(2 * lo + 1):
            d = acc.get((lo, k), {})
            widths = sorted(d, reverse=True)
            if widths and widths[0] == W:
                cur = d[W]
                widths = widths[1:]
            else:
                cur = jnp.zeros((W, nb), xt.dtype)
            for w in widths:
                cur = jnp.concatenate([cur[:w, :] + d[w], cur[w:, :]], axis=0)
            blocks.append(cur)
    o_ref[...] = jnp.concatenate(blocks, axis=0).T


def _tc_call(x, kc, mc, NB=2048):
    n = x.shape[0]
    grid = (n // NB,)
    return pl.pallas_call(
        _body,
        grid=grid,
        compiler_params=pltpu.CompilerParams(
            dimension_semantics=("arbitrary",),
        ),
        in_specs=[
            pl.BlockSpec((NB, _DIM_IN), lambda i: (i, 0)),
            pl.BlockSpec((224, 1), lambda i: (0, 0)),
            pl.BlockSpec((864, 1), lambda i: (0, 0)),
        ],
        out_specs=pl.BlockSpec((NB, _DIM_OUT), lambda i: (i, 0)),
        out_shape=jax.ShapeDtypeStruct((n, _DIM_OUT), x.dtype),
    )(x, kc, mc)


_NW = 32
_L = 16


def _norm_key(l1, l2, i, j):
    return (l1, l2, i, j) if (l1, l2, i, j) <= (l2, l1, j, i) else (l2, l1, j, i)


def _build_sc_schedule():
    uses = {}
    for (lo, l1, l2, deg, mc_off, nzs) in _TERMS:
        for (k, i, j, v) in nzs:
            key = _norm_key(l1, l2, i, j)
            uses[key] = uses.get(key, 0) + 1
    prod_base = {}
    tot = 0
    for (lo, l1, l2, deg, mc_off, nzs) in _TERMS:
        for (k, i, j, v) in nzs:
            key = _norm_key(l1, l2, i, j)
            if key not in prod_base:
                prod_base[key] = tot
                tot += deg
    def xoff(l, m):
        return _IN_OFF[l] + m * _METADATA_IN[l]
    p1 = []
    for (l1, l2, i, j), b in prod_base.items():
        deg = min(_METADATA_IN[l1], _METADATA_IN[l2])
        for cc in range(deg // _L):
            p1.append((b + cc * _L, xoff(l1, i) + cc * _L, xoff(l2, j) + cc * _L))
    out_off = {}
    off = 0
    for lo in range(_LMAX_OUT + 1):
        W = _META_OUT[lo]
        for k in range(2 * lo + 1):
            out_off[(lo, k)] = off
            off += W
    ochunks = {}
    tcoef = []
    for (lo, l1, l2, deg, mc_off, nzs) in _TERMS:
        for (k, i, j, v) in nzs:
            key = _norm_key(l1, l2, i, j)
            for cc in range(deg // _L):
                ci = len(tcoef) * _L
                tcoef.append((mc_off + cc * _L, 0.5 * v))
                if key in prod_base:
                    op = ('p', prod_base[key] + cc * _L, ci)
                else:
                    op = ('x', xoff(l1, i) + cc * _L, xoff(l2, j) + cc * _L, ci)
                ochunks.setdefault((lo, k, cc), []).append(op)
    ch = 0
    for l, nc in enumerate(_METADATA_IN):
        cp = min(nc, _META_OUT[l])
        for m in range(2 * l + 1):
            for cc in range(cp // _L):
                ochunks.setdefault((l, m, cc), []).append(
                    ('kp', xoff(l, m) + cc * _L, ch + cc * _L))
        ch += nc
    sched = []
    for lo in range(_LMAX_OUT + 1):
        W = _META_OUT[lo]
        for k in range(2 * lo + 1):
            for cc in range(W // _L):
                sched.append((out_off[(lo, k)] + cc * _L,
                              ochunks.get((lo, k, cc), [])))
    return p1, sched, tot, tcoef


_P1, _SCHED, _PROD_TOT, _TCOEF = _build_sc_schedule()


def _sc_call(x, kc, mc):
    n = x.shape[0]
    per_w = n // _NW
    mesh = plsc.VectorSubcoreMesh(core_axis_name="c", subcore_axis_name="s")

    @functools.partial(
        pl.kernel,
        out_type=jax.ShapeDtypeStruct((n, _DIM_OUT), jnp.float32),
        mesh=mesh,
        scratch_types=[
            pltpu.VMEM((1, _DIM_IN), jnp.float32),
            pltpu.VMEM((1, _DIM_OUT), jnp.float32),
            pltpu.VMEM((224,), jnp.float32),
            pltpu.VMEM((864,), jnp.float32),
            pltpu.VMEM((_PROD_TOT,), jnp.float32),
        ],
    )
    def sc_kernel(x_hbm, kc_hbm, mc_hbm, out_hbm, xbuf, obuf, kcbuf, mcbuf, pbuf):
        wid = lax.axis_index("s") * 2 + lax.axis_index("c")
        base = wid * per_w
        pltpu.sync_copy(kc_hbm, kcbuf)
        pltpu.sync_copy(mc_hbm, mcbuf)

        def step(it, carry):
            nb = base + it
            pltpu.sync_copy(x_hbm.at[pl.ds(nb, 1)], xbuf)
            for (pb, o1, o2) in _P1:
                pbuf[pl.ds(pb, _L)] = (xbuf[0, pl.ds(o1, _L)]
                                       * xbuf[0, pl.ds(o2, _L)])
            for (oo, ops) in _SCHED:
                acc = None
                for op in ops:
                    if op[0] == 'p':
                        _, pb, ci = op
                        mo, s = _TCOEF[ci // _L]
                        c = pbuf[pl.ds(pb, _L)] * (mcbuf[pl.ds(mo, _L)] * s)
                    else:
                        _, xo, ko = op
                        c = xbuf[0, pl.ds(xo, _L)] * kcbuf[pl.ds(ko, _L)]
                    acc = c if acc is None else acc + c
                if acc is None:
                    acc = jnp.zeros((_L,), jnp.float32)
                obuf[0, pl.ds(oo, _L)] = acc
            pltpu.sync_copy(obuf, out_hbm.at[pl.ds(nb, 1)])
            return carry

        lax.fori_loop(0, per_w, step, 0)

    return sc_kernel(x, kc, mc)


def kernel(x, keep_coeff, mix_coeff):
    return _tc_call(x, keep_coeff.reshape(-1, 1), mix_coeff.reshape(-1, 1),
                    NB=2048)
